# baseline (speedup 1.0000x reference)
import jax
import jax.numpy as jnp
from jax.experimental import pallas as pl

S = 4096
E = 8
CAP = 1024
TOK_BLOCK = 256


def _cumsum_rows(x):
    n = x.shape[0]
    k = 1
    while k < n:
        shifted = jnp.concatenate(
            [jnp.zeros((k, x.shape[1]), x.dtype), x[: n - k]], axis=0)
        x = x + shifted
        k *= 2
    return x


def _route_kernel(x_ref, wf_ref, fi_ref):
    x = x_ref[...]
    m = jnp.max(x, axis=-1, keepdims=True)
    ex = jnp.exp(x - m)
    probs = ex / jnp.sum(ex, axis=-1, keepdims=True)

    eio = jax.lax.broadcasted_iota(jnp.int32, (S, E), 1)
    p1 = jnp.max(probs, axis=-1, keepdims=True)
    e1 = jnp.min(jnp.where(probs == p1, eio, E), axis=-1, keepdims=True)
    mask1 = eio == e1
    pe = jnp.where(mask1, -jnp.inf, probs)
    p2 = jnp.max(pe, axis=-1, keepdims=True)
    e2 = jnp.min(jnp.where(pe == p2, eio, E), axis=-1, keepdims=True)
    mask2 = eio == e2

    m1i = mask1.astype(jnp.int32)
    m2i = mask2.astype(jnp.int32)
    c1 = _cumsum_rows(m1i)
    rank1 = c1 - 1
    count1 = c1[S - 1:S, :]
    rank2 = _cumsum_rows(m2i) - 1 + count1

    keep1 = mask1 & (rank1 < CAP)
    keep2 = mask2 & (rank2 < CAP)
    w1 = jnp.sum(jnp.where(keep1, probs, 0.0), axis=-1, keepdims=True)
    w2 = jnp.sum(jnp.where(keep2, probs, 0.0), axis=-1, keepdims=True)
    r1 = jnp.sum(jnp.where(keep1, rank1, 0), axis=-1, keepdims=True)
    r2 = jnp.sum(jnp.where(keep2, rank2, 0), axis=-1, keepdims=True)

    wf_ref[...] = jnp.concatenate([w1, w2], axis=1)
    fi_ref[...] = jnp.concatenate([e1 * CAP + r1, e2 * CAP + r2], axis=1)


def _expand_kernel(wf_ref, fi_ref, out_ref, msk_ref):
    w1 = wf_ref[:, 0:1]
    w2 = wf_ref[:, 1:2]
    f1 = fi_ref[:, 0:1]
    f2 = fi_ref[:, 1:2]
    cols = jax.lax.broadcasted_iota(jnp.int32, (TOK_BLOCK, E * CAP), 1)
    out = jnp.where(cols == f1, w1, 0.0) + jnp.where(cols == f2, w2, 0.0)
    out_ref[...] = out
    msk_ref[...] = out != 0.0


def kernel(inputs):
    wf, fi = pl.pallas_call(
        _route_kernel,
        out_shape=(
            jax.ShapeDtypeStruct((S, 2), jnp.float32),
            jax.ShapeDtypeStruct((S, 2), jnp.int32),
        ),
    )(inputs)

    nblk = S // TOK_BLOCK
    cb, msk = pl.pallas_call(
        _expand_kernel,
        grid=(nblk,),
        in_specs=[
            pl.BlockSpec((TOK_BLOCK, 2), lambda i: (i, 0)),
            pl.BlockSpec((TOK_BLOCK, 2), lambda i: (i, 0)),
        ],
        out_specs=(
            pl.BlockSpec((TOK_BLOCK, E * CAP), lambda i: (i, 0)),
            pl.BlockSpec((TOK_BLOCK, E * CAP), lambda i: (i, 0)),
        ),
        out_shape=(
            jax.ShapeDtypeStruct((S, E * CAP), jnp.float32),
            jax.ShapeDtypeStruct((S, E * CAP), jnp.bool_),
        ),
    )(wf, fi)
    return (cb.reshape(S, E, CAP), msk.reshape(S, E, CAP))


# speedup vs baseline: 1.6865x; 1.6865x over previous
import jax
import jax.numpy as jnp
from jax.experimental import pallas as pl

S = 4096
E = 8
CAP = 1024
ROW_BLOCK = 2048


def _cumsum_rows(x):
    n = x.shape[0]
    k = 1
    while k < n:
        shifted = jnp.concatenate(
            [jnp.zeros((k, x.shape[1]), x.dtype), x[: n - k]], axis=0)
        x = x + shifted
        k *= 2
    return x


def _route_kernel(x_ref, qr_ref, qw_ref):
    x = x_ref[...]
    m = jnp.max(x, axis=-1, keepdims=True)
    ex = jnp.exp(x - m)
    probs = ex / jnp.sum(ex, axis=-1, keepdims=True)

    eio = jax.lax.broadcasted_iota(jnp.int32, (S, E), 1)
    p1 = jnp.max(probs, axis=-1, keepdims=True)
    e1 = jnp.min(jnp.where(probs == p1, eio, E), axis=-1, keepdims=True)
    mask1 = eio == e1
    pe = jnp.where(mask1, -jnp.inf, probs)
    p2 = jnp.max(pe, axis=-1, keepdims=True)
    e2 = jnp.min(jnp.where(pe == p2, eio, E), axis=-1, keepdims=True)
    mask2 = eio == e2

    c1 = _cumsum_rows(mask1.astype(jnp.int32))
    rank1 = c1 - 1
    count1 = c1[S - 1:S, :]
    rank2 = _cumsum_rows(mask2.astype(jnp.int32)) - 1 + count1

    keep1 = mask1 & (rank1 < CAP)
    keep2 = mask2 & (rank2 < CAP)
    qr_ref[...] = jnp.where(keep1, rank1, jnp.where(keep2, rank2, -1))
    qw_ref[...] = probs


def _expand_kernel(qr_ref, qw_ref, out_ref, msk_ref):
    qr = qr_ref[...]
    qw = qw_ref[...]
    cols = jax.lax.broadcasted_iota(jnp.int32, (ROW_BLOCK, CAP), 1)
    out = jnp.where(cols == qr, qw, 0.0)
    out_ref[...] = out
    msk_ref[...] = out != 0.0


def kernel(inputs):
    qr, qw = pl.pallas_call(
        _route_kernel,
        out_shape=(
            jax.ShapeDtypeStruct((S, E), jnp.int32),
            jax.ShapeDtypeStruct((S, E), jnp.float32),
        ),
    )(inputs)

    qr8 = qr.reshape(S * E, 1)
    qw8 = qw.reshape(S * E, 1)

    nblk = (S * E) // ROW_BLOCK
    cb, msk = pl.pallas_call(
        _expand_kernel,
        grid=(nblk,),
        in_specs=[
            pl.BlockSpec((ROW_BLOCK, 1), lambda i: (i, 0)),
            pl.BlockSpec((ROW_BLOCK, 1), lambda i: (i, 0)),
        ],
        out_specs=(
            pl.BlockSpec((ROW_BLOCK, CAP), lambda i: (i, 0)),
            pl.BlockSpec((ROW_BLOCK, CAP), lambda i: (i, 0)),
        ),
        out_shape=(
            jax.ShapeDtypeStruct((S * E, CAP), jnp.float32),
            jax.ShapeDtypeStruct((S * E, CAP), jnp.bool_),
        ),
    )(qr8, qw8)
    return (cb.reshape(S, E, CAP), msk.reshape(S, E, CAP))
